# baseline (speedup 1.0000x reference)
import functools

import jax
import jax.numpy as jnp
from jax.experimental import pallas as pl


def _l2_band_kernel(lhs_ref, rhs_ref, out_ref, *, tail):
    bb, N, N_ = out_ref.shape
    out_ref[...] = jnp.full(out_ref.shape, -1000000000.0, dtype=out_ref.dtype)
    diff = lhs_ref[...] - rhs_ref[...]
    sim = jnp.sqrt(jnp.sum(diff * diff, axis=-1))
    row = jax.lax.broadcasted_iota(jnp.int32, (bb, N, tail), 1)
    col = jax.lax.broadcasted_iota(jnp.int32, (bb, N, tail), 2)
    mask = col == row + (tail - N)
    out_ref[:, :, N_ - tail:] = jnp.where(
        mask, sim[:, :, None], jnp.float32(-1000000000.0)
    )


def kernel(lhs, rhs):
    B, N, dim = lhs.shape
    N_ = rhs.shape[1]
    bb = 2
    tail = 128
    tail_block_idx = N_ // N - 1

    body = functools.partial(_l2_band_kernel, tail=tail)
    return pl.pallas_call(
        body,
        grid=(B // bb,),
        in_specs=[
            pl.BlockSpec((bb, N, dim), lambda j: (j, 0, 0)),
            pl.BlockSpec((bb, N, dim), lambda j: (j, tail_block_idx, 0)),
        ],
        out_specs=pl.BlockSpec((bb, N, N_), lambda j: (j, 0, 0)),
        out_shape=jax.ShapeDtypeStruct((B, N, N_), lhs.dtype),
    )(lhs, rhs)


# speedup vs baseline: 1.2758x; 1.2758x over previous
import functools

import jax
import jax.numpy as jnp
from jax.experimental import pallas as pl


def _l2_band_kernel(lhs_ref, rhs_ref, out_ref, *, tail):
    bb, N, N_ = out_ref.shape
    out_ref[...] = jnp.full(out_ref.shape, -1000000000.0, dtype=out_ref.dtype)
    diff = lhs_ref[...] - rhs_ref[...]
    sim = jnp.sqrt(jnp.sum(diff * diff, axis=-1))
    row = jax.lax.broadcasted_iota(jnp.int32, (bb, N, tail), 1)
    col = jax.lax.broadcasted_iota(jnp.int32, (bb, N, tail), 2)
    mask = col == row + (tail - N)
    out_ref[:, :, N_ - tail:] = jnp.where(
        mask, sim[:, :, None], jnp.float32(-1000000000.0)
    )


def kernel(lhs, rhs):
    B, N, dim = lhs.shape
    N_ = rhs.shape[1]
    bb = 4
    tail = 128
    tail_block_idx = N_ // N - 1

    body = functools.partial(_l2_band_kernel, tail=tail)
    return pl.pallas_call(
        body,
        grid=(B // bb,),
        in_specs=[
            pl.BlockSpec((bb, N, dim), lambda j: (j, 0, 0)),
            pl.BlockSpec((bb, N, dim), lambda j: (j, tail_block_idx, 0)),
        ],
        out_specs=pl.BlockSpec((bb, N, N_), lambda j: (j, 0, 0)),
        out_shape=jax.ShapeDtypeStruct((B, N, N_), lhs.dtype),
    )(lhs, rhs)
